# speedup vs baseline: 1.0752x; 1.0752x over previous
import functools

import jax
import jax.numpy as jnp
import numpy as np
from jax.experimental import pallas as pl
from jax.experimental.pallas import tpu as pltpu

N_KNN = 16
_BN_SCALE = 1.0 / np.sqrt(1.0 + 1e-5)


def _sqdist(src, dst):
    d = -2.0 * jnp.einsum('bsd,bnd->bsn', src, dst)
    d = d + jnp.sum(src ** 2, axis=-1)[:, :, None]
    d = d + jnp.sum(dst ** 2, axis=-1)[:, None, :]
    return d


def _knn(k, xyz, new_xyz):
    d = _sqdist(new_xyz, xyz)
    _, idx = jax.lax.top_k(-d, k)
    return idx


def _fps(xyz, npoint):
    B, N, _ = xyz.shape
    def body(i, state):
        dist, farthest, idxs = state
        idxs = idxs.at[:, i].set(farthest)
        centroid = jnp.take_along_axis(xyz, farthest[:, None, None], axis=1)
        d = jnp.sum((xyz - centroid) ** 2, axis=-1)
        dist = jnp.minimum(dist, d)
        farthest = jnp.argmax(dist, axis=-1).astype(jnp.int32)
        return dist, farthest, idxs
    state = (jnp.full((B, N), 1e10, dtype=jnp.float32),
             jnp.zeros((B,), dtype=jnp.int32),
             jnp.zeros((B, npoint), dtype=jnp.int32))
    _, _, idxs = jax.lax.fori_loop(0, npoint, body, state)
    return idxs


def _group(points, idx):
    B, C, N = points.shape
    S, K = idx.shape[1], idx.shape[2]
    flat = jnp.broadcast_to(idx.reshape(B, 1, S * K), (B, C, S * K))
    return jnp.take_along_axis(points, flat, axis=2).reshape(B, C, S, K)


def _conv1d(x, w, b):
    return jnp.einsum('oc,bcn->bon', w, x) + b[None, :, None]


def _conv2d(x, w, b):
    return jnp.einsum('oc,bcnk->bonk', w, x) + b[None, :, None, None]


def _bn2d(x, g, b):
    return x * (g * _BN_SCALE)[None, :, None, None] + b[None, :, None, None]


def _sa_module(xyz, points, npoint, k, p, pre):
    B = xyz.shape[0]
    xyz_t = jnp.transpose(xyz, (0, 2, 1))
    fps_idx = _fps(xyz_t, npoint)
    new_xyz = jnp.take_along_axis(
        xyz, jnp.broadcast_to(fps_idx[:, None, :], (B, 3, npoint)), axis=2)
    idx = _knn(k, xyz_t, jnp.transpose(new_xyz, (0, 2, 1)))
    grouped_xyz = _group(xyz, idx) - new_xyz[:, :, :, None]
    grouped_points = _group(points, idx)
    new_points = jnp.concatenate([grouped_xyz, grouped_points], axis=1)
    h = jax.nn.relu(_conv2d(new_points, p[pre + '_c0_w'], p[pre + '_c0_b']))
    h = _conv2d(h, p[pre + '_c1_w'], p[pre + '_c1_b'])
    return new_xyz, jnp.max(h, axis=3), idx


def _transformer(x, pos, p, pre):
    identity = x
    x = _conv1d(x, p[pre + '_start_w'], p[pre + '_start_b'])
    pos_t = jnp.transpose(pos, (0, 2, 1))
    idx = _knn(N_KNN, pos_t, pos_t)
    key = _group(_conv1d(x, p[pre + '_key_w'], p[pre + '_key_b']), idx)
    value = _conv1d(x, p[pre + '_value_w'], p[pre + '_value_b'])
    query = _conv1d(x, p[pre + '_query_w'], p[pre + '_query_b'])
    qk_rel = query[:, :, :, None] - key
    pos_rel = pos[:, :, :, None] - _group(pos, idx)
    pe = _conv2d(pos_rel, p[pre + '_pos0_w'], p[pre + '_pos0_b'])
    pe = jax.nn.relu(_bn2d(pe, p[pre + '_pos_bn_g'], p[pre + '_pos_bn_b']))
    pe = _conv2d(pe, p[pre + '_pos1_w'], p[pre + '_pos1_b'])
    attn = _conv2d(qk_rel + pe, p[pre + '_attn0_w'], p[pre + '_attn0_b'])
    attn = jax.nn.relu(_bn2d(attn, p[pre + '_attn_bn_g'], p[pre + '_attn_bn_b']))
    attn = _conv2d(attn, p[pre + '_attn1_w'], p[pre + '_attn1_b'])
    attn = jax.nn.softmax(attn, axis=-1)
    value = value[:, :, :, None] + pe
    agg = jnp.sum(attn * value, axis=-1)
    y = _conv1d(agg, p[pre + '_end_w'], p[pre + '_end_b'])
    return y + identity


def _sa3_body(xyz_ref, pts_ref, w0x_ref, w0p_ref, b0_ref, w1_ref, b1_ref,
              out_ref):
    xyz = xyz_ref[0]
    pts = pts_ref[0]
    h = (jnp.dot(w0x_ref[...], xyz, preferred_element_type=jnp.float32)
         + jnp.dot(w0p_ref[...], pts, preferred_element_type=jnp.float32)
         + b0_ref[...][:, None])
    h = jnp.maximum(h, 0.0)
    y = jnp.dot(w1_ref[...], h, preferred_element_type=jnp.float32)
    y = y + b1_ref[...][:, None]
    out_ref[0] = jnp.max(y, axis=1, keepdims=True)


def _sa3_pallas(l2_xyz, l2_points, w0, b0, w1, b1):
    B = l2_xyz.shape[0]
    w0x = w0[:, :3]
    w0p = w0[:, 3:]
    return pl.pallas_call(
        _sa3_body,
        grid=(B,),
        in_specs=[
            pl.BlockSpec((1, 3, 64), lambda b: (b, 0, 0)),
            pl.BlockSpec((1, 128, 64), lambda b: (b, 0, 0)),
            pl.BlockSpec((256, 3), lambda b: (0, 0)),
            pl.BlockSpec((256, 128), lambda b: (0, 0)),
            pl.BlockSpec((256,), lambda b: (0,)),
            pl.BlockSpec((512, 256), lambda b: (0, 0)),
            pl.BlockSpec((512,), lambda b: (0,)),
        ],
        out_specs=pl.BlockSpec((1, 512, 1), lambda b: (b, 0, 0)),
        out_shape=jax.ShapeDtypeStruct((B, 512, 1), jnp.float32),
    )(l2_xyz, l2_points, w0x, w0p, b0, w1, b1)


def kernel(point_cloud, sa1_c0_w, sa1_c0_b, sa1_c1_w, sa1_c1_b, t1_start_w, t1_start_b, t1_key_w, t1_key_b, t1_query_w, t1_query_b, t1_value_w, t1_value_b, t1_pos0_w, t1_pos0_b, t1_pos_bn_g, t1_pos_bn_b, t1_pos1_w, t1_pos1_b, t1_attn0_w, t1_attn0_b, t1_attn_bn_g, t1_attn_bn_b, t1_attn1_w, t1_attn1_b, t1_end_w, t1_end_b, sa2_c0_w, sa2_c0_b, sa2_c1_w, sa2_c1_b, t2_start_w, t2_start_b, t2_key_w, t2_key_b, t2_query_w, t2_query_b, t2_value_w, t2_value_b, t2_pos0_w, t2_pos0_b, t2_pos_bn_g, t2_pos_bn_b, t2_pos1_w, t2_pos1_b, t2_attn0_w, t2_attn0_b, t2_attn_bn_g, t2_attn_bn_b, t2_attn1_w, t2_attn1_b, t2_end_w, t2_end_b, sa3_c0_w, sa3_c0_b, sa3_c1_w, sa3_c1_b):
    p = dict(locals())
    del p['point_cloud']
    l1_xyz, l1_points, _ = _sa_module(point_cloud, point_cloud, 256, 16, p, 'sa1')
    l1_points = _transformer(l1_points, l1_xyz, p, 't1')
    l2_xyz, l2_points, _ = _sa_module(l1_xyz, l1_points, 64, 16, p, 'sa2')
    l2_points = _transformer(l2_points, l2_xyz, p, 't2')
    return _sa3_pallas(l2_xyz, l2_points, sa3_c0_w, sa3_c0_b, sa3_c1_w, sa3_c1_b)


# speedup vs baseline: 118.6955x; 110.3978x over previous
import functools

import jax
import jax.numpy as jnp
import numpy as np
from jax.experimental import pallas as pl
from jax.experimental.pallas import tpu as pltpu

N_KNN = 16
_BN_SCALE = np.float32(1.0 / np.sqrt(1.0 + 1e-5))
_F32 = jnp.float32
_BIG = np.float32(3.0e38)


def _fps_body(xyz_ref, out_ref, *, npoint):
    x = xyz_ref[:, 0, :]
    y = xyz_ref[:, 1, :]
    z = xyz_ref[:, 2, :]
    B, N = x.shape
    lane = jax.lax.broadcasted_iota(jnp.int32, (B, N), 1)
    lane_s = jax.lax.broadcasted_iota(jnp.int32, (B, npoint), 1)

    def body(i, carry):
        dist, far, nx, ny, nz = carry
        oh = lane == far
        cx = jnp.sum(jnp.where(oh, x, 0.0), axis=1, keepdims=True)
        cy = jnp.sum(jnp.where(oh, y, 0.0), axis=1, keepdims=True)
        cz = jnp.sum(jnp.where(oh, z, 0.0), axis=1, keepdims=True)
        sel = lane_s == i
        nx = jnp.where(sel, cx, nx)
        ny = jnp.where(sel, cy, ny)
        nz = jnp.where(sel, cz, nz)
        dx = x - cx
        dy = y - cy
        dz = z - cz
        d = (dx * dx + dy * dy) + dz * dz
        dist = jnp.minimum(dist, d)
        m = jnp.max(dist, axis=1, keepdims=True)
        far = jnp.min(jnp.where(dist == m, lane, N), axis=1, keepdims=True)
        return dist, far, nx, ny, nz

    zs = jnp.zeros((B, npoint), dtype=_F32)
    _, _, nx, ny, nz = jax.lax.fori_loop(
        0, npoint, body,
        (jnp.full((B, N), 1e10, dtype=_F32), jnp.zeros((B, 1), jnp.int32),
         zs, zs, zs))
    out_ref[:, 0, :] = nx
    out_ref[:, 1, :] = ny
    out_ref[:, 2, :] = nz


def _fps_pallas(xyz, npoint):
    B, _, N = xyz.shape
    return pl.pallas_call(
        functools.partial(_fps_body, npoint=npoint),
        out_shape=jax.ShapeDtypeStruct((B, 3, npoint), _F32),
    )(xyz)


def _sa_body(xyz_ref, xyzt_ref, pts_ref, new_ref,
             w0x_ref, w0p_ref, b0_ref, w1_ref, b1_ref, out_ref):
    xyz = xyz_ref[0]
    xyz_t = xyzt_ref[0]
    pts = pts_ref[0]
    new = new_ref[0]
    N = xyz_t.shape[0]
    S = new.shape[1]
    Cout = out_ref.shape[1]

    xt0, xt1, xt2 = xyz_t[:, 0:1], xyz_t[:, 1:2], xyz_t[:, 2:3]
    q0, q1, q2 = new[0:1, :], new[1:2, :], new[2:3, :]
    sqq = (q0 * q0 + q1 * q1) + q2 * q2
    sqx = (xt0 * xt0 + xt1 * xt1) + xt2 * xt2
    dot = jnp.dot(xyz_t, new, preferred_element_type=_F32)
    dt = (-2.0 * dot + sqq) + sqx
    sub = jax.lax.broadcasted_iota(jnp.int32, (N, S), 0)

    w0x = w0x_ref[...]
    w0p = w0p_ref[...]
    b0 = b0_ref[...]
    w1 = w1_ref[...]
    b1 = b1_ref[...]

    run = jnp.full((Cout, S), -_BIG, dtype=_F32)
    for _ in range(N_KNN):
        m = jnp.min(dt, axis=0, keepdims=True)
        idxk = jnp.min(jnp.where(dt == m, sub, N), axis=0,
                       keepdims=True)
        oh = sub == idxk
        ohf = jnp.where(oh, np.float32(1.0), np.float32(0.0))
        gx = jnp.dot(xyz, ohf, preferred_element_type=_F32)
        gp = jnp.dot(pts, ohf, preferred_element_type=_F32)
        h = (jnp.dot(w0x, gx - new, preferred_element_type=_F32)
             + jnp.dot(w0p, gp, preferred_element_type=_F32) + b0)
        h = jnp.maximum(h, 0.0)
        yk = jnp.dot(w1, h, preferred_element_type=_F32) + b1
        run = jnp.maximum(run, yk)
        dt = jnp.where(oh, _BIG, dt)
    out_ref[0] = run


def _sa_pallas(xyz, xyz_t, pts, new_xyz, w0, b0, w1, b1):
    B, _, N = xyz.shape
    C = pts.shape[1]
    S = new_xyz.shape[2]
    Cout = w1.shape[0]
    Chid = w0.shape[0]
    return pl.pallas_call(
        _sa_body,
        grid=(B,),
        in_specs=[
            pl.BlockSpec((1, 3, N), lambda b: (b, 0, 0)),
            pl.BlockSpec((1, N, 3), lambda b: (b, 0, 0)),
            pl.BlockSpec((1, C, N), lambda b: (b, 0, 0)),
            pl.BlockSpec((1, 3, S), lambda b: (b, 0, 0)),
            pl.BlockSpec((Chid, 3), lambda b: (0, 0)),
            pl.BlockSpec((Chid, C), lambda b: (0, 0)),
            pl.BlockSpec((Chid, 1), lambda b: (0, 0)),
            pl.BlockSpec((Cout, Chid), lambda b: (0, 0)),
            pl.BlockSpec((Cout, 1), lambda b: (0, 0)),
        ],
        out_specs=pl.BlockSpec((1, Cout, S), lambda b: (b, 0, 0)),
        out_shape=jax.ShapeDtypeStruct((B, Cout, S), _F32),
    )(xyz, xyz_t, pts, new_xyz, w0[:, :3], w0[:, 3:], b0[:, None],
      w1, b1[:, None])


def _t_body(x_ref, pos_ref, post_ref,
            wst_ref, bst_ref, wk_ref, bk_ref, wq_ref, bq_ref, wv_ref, bv_ref,
            wp0_ref, bp0_ref, wp1_ref, bp1_ref,
            wa0_ref, ba0_ref, wa1_ref, ba1_ref,
            wend_ref, bend_ref, out_ref):
    x_in = x_ref[0]
    pos = pos_ref[0]
    pos_t = post_ref[0]
    S = pos.shape[1]
    K = N_KNN

    x = jnp.dot(wst_ref[...], x_in, preferred_element_type=_F32) + bst_ref[...]
    keya = jnp.dot(wk_ref[...], x, preferred_element_type=_F32) + bk_ref[...]
    val = jnp.dot(wv_ref[...], x, preferred_element_type=_F32) + bv_ref[...]
    qry = jnp.dot(wq_ref[...], x, preferred_element_type=_F32) + bq_ref[...]

    p0, p1, p2 = pos[0:1, :], pos[1:2, :], pos[2:3, :]
    pt0, pt1, pt2 = pos_t[:, 0:1], pos_t[:, 1:2], pos_t[:, 2:3]
    sqq = (p0 * p0 + p1 * p1) + p2 * p2
    sqx = (pt0 * pt0 + pt1 * pt1) + pt2 * pt2
    dot = jnp.dot(pos_t, pos, preferred_element_type=_F32)
    dt = (-2.0 * dot + sqq) + sqx
    sub = jax.lax.broadcasted_iota(jnp.int32, (S, S), 0)

    qks = []
    prs = []
    for _ in range(K):
        m = jnp.min(dt, axis=0, keepdims=True)
        idxk = jnp.min(jnp.where(dt == m, sub, S), axis=0, keepdims=True)
        oh = sub == idxk
        ohf = jnp.where(oh, np.float32(1.0), np.float32(0.0))
        kg = jnp.dot(keya, ohf, preferred_element_type=_F32)
        pg = jnp.dot(pos, ohf, preferred_element_type=_F32)
        qks.append(qry - kg)
        prs.append(pos - pg)
        dt = jnp.where(oh, _BIG, dt)
    qk = jnp.concatenate(qks, axis=1)
    prel = jnp.concatenate(prs, axis=1)

    pe = jnp.dot(wp0_ref[...], prel, preferred_element_type=_F32) + bp0_ref[...]
    pe = jnp.maximum(pe, 0.0)
    pe = jnp.dot(wp1_ref[...], pe, preferred_element_type=_F32) + bp1_ref[...]

    a = jnp.dot(wa0_ref[...], qk + pe, preferred_element_type=_F32) + ba0_ref[...]
    a = jnp.maximum(a, 0.0)
    attn = jnp.dot(wa1_ref[...], a, preferred_element_type=_F32) + ba1_ref[...]

    m0 = attn[:, 0:S]
    for k in range(1, K):
        m0 = jnp.maximum(m0, attn[:, k * S:(k + 1) * S])
    es = []
    esum = jnp.zeros_like(m0)
    for k in range(K):
        e = jnp.exp(attn[:, k * S:(k + 1) * S] - m0)
        es.append(e)
        esum = esum + e
    agg = jnp.zeros_like(m0)
    for k in range(K):
        agg = agg + (es[k] / esum) * (val + pe[:, k * S:(k + 1) * S])

    y = jnp.dot(wend_ref[...], agg, preferred_element_type=_F32) + bend_ref[...]
    out_ref[0] = y + x_in


def _t_pallas(x, pos, pos_t, p, pre):
    B, Cin, S = x.shape
    w = {n: p[pre + n] for n in (
        '_start_w', '_key_w', '_query_w', '_value_w', '_pos0_w', '_pos1_w',
        '_attn0_w', '_attn1_w', '_end_w')}
    p0w = w['_pos0_w'] * (p[pre + '_pos_bn_g'] * _BN_SCALE)[:, None]
    p0b = (p[pre + '_pos0_b'] * p[pre + '_pos_bn_g'] * _BN_SCALE
           + p[pre + '_pos_bn_b'])
    a0w = w['_attn0_w'] * (p[pre + '_attn_bn_g'] * _BN_SCALE)[:, None]
    a0b = (p[pre + '_attn0_b'] * p[pre + '_attn_bn_g'] * _BN_SCALE
           + p[pre + '_attn_bn_b'])

    def wspec(a):
        return pl.BlockSpec(a.shape, lambda b: tuple(0 for _ in a.shape))

    ws = [w['_start_w'], p[pre + '_start_b'][:, None],
          w['_key_w'], p[pre + '_key_b'][:, None],
          w['_query_w'], p[pre + '_query_b'][:, None],
          w['_value_w'], p[pre + '_value_b'][:, None],
          p0w, p0b[:, None],
          w['_pos1_w'], p[pre + '_pos1_b'][:, None],
          a0w, a0b[:, None],
          w['_attn1_w'], p[pre + '_attn1_b'][:, None],
          w['_end_w'], p[pre + '_end_b'][:, None]]
    return pl.pallas_call(
        _t_body,
        grid=(B,),
        in_specs=[
            pl.BlockSpec((1, Cin, S), lambda b: (b, 0, 0)),
            pl.BlockSpec((1, 3, S), lambda b: (b, 0, 0)),
            pl.BlockSpec((1, S, 3), lambda b: (b, 0, 0)),
        ] + [wspec(a) for a in ws],
        out_specs=pl.BlockSpec((1, Cin, S), lambda b: (b, 0, 0)),
        out_shape=jax.ShapeDtypeStruct((B, Cin, S), _F32),
    )(x, pos, pos_t, *ws)


def _sa3_body(xyz_ref, pts_ref, w0x_ref, w0p_ref, b0_ref, w1_ref, b1_ref,
              out_ref):
    xyz = xyz_ref[0]
    pts = pts_ref[0]
    h = (jnp.dot(w0x_ref[...], xyz, preferred_element_type=_F32)
         + jnp.dot(w0p_ref[...], pts, preferred_element_type=_F32)
         + b0_ref[...])
    h = jnp.maximum(h, 0.0)
    y = jnp.dot(w1_ref[...], h, preferred_element_type=_F32)
    y = y + b1_ref[...]
    out_ref[0] = jnp.max(y, axis=1, keepdims=True)


def _sa3_pallas(l2_xyz, l2_points, w0, b0, w1, b1):
    B = l2_xyz.shape[0]
    return pl.pallas_call(
        _sa3_body,
        grid=(B,),
        in_specs=[
            pl.BlockSpec((1, 3, 64), lambda b: (b, 0, 0)),
            pl.BlockSpec((1, 128, 64), lambda b: (b, 0, 0)),
            pl.BlockSpec((256, 3), lambda b: (0, 0)),
            pl.BlockSpec((256, 128), lambda b: (0, 0)),
            pl.BlockSpec((256, 1), lambda b: (0, 0)),
            pl.BlockSpec((512, 256), lambda b: (0, 0)),
            pl.BlockSpec((512, 1), lambda b: (0, 0)),
        ],
        out_specs=pl.BlockSpec((1, 512, 1), lambda b: (b, 0, 0)),
        out_shape=jax.ShapeDtypeStruct((B, 512, 1), _F32),
    )(l2_xyz, l2_points, w0[:, :3], w0[:, 3:], b0[:, None], w1, b1[:, None])


def kernel(point_cloud, sa1_c0_w, sa1_c0_b, sa1_c1_w, sa1_c1_b, t1_start_w, t1_start_b, t1_key_w, t1_key_b, t1_query_w, t1_query_b, t1_value_w, t1_value_b, t1_pos0_w, t1_pos0_b, t1_pos_bn_g, t1_pos_bn_b, t1_pos1_w, t1_pos1_b, t1_attn0_w, t1_attn0_b, t1_attn_bn_g, t1_attn_bn_b, t1_attn1_w, t1_attn1_b, t1_end_w, t1_end_b, sa2_c0_w, sa2_c0_b, sa2_c1_w, sa2_c1_b, t2_start_w, t2_start_b, t2_key_w, t2_key_b, t2_query_w, t2_query_b, t2_value_w, t2_value_b, t2_pos0_w, t2_pos0_b, t2_pos_bn_g, t2_pos_bn_b, t2_pos1_w, t2_pos1_b, t2_attn0_w, t2_attn0_b, t2_attn_bn_g, t2_attn_bn_b, t2_attn1_w, t2_attn1_b, t2_end_w, t2_end_b, sa3_c0_w, sa3_c0_b, sa3_c1_w, sa3_c1_b):
    p = dict(locals())
    del p['point_cloud']

    pc_t = jnp.transpose(point_cloud, (0, 2, 1))
    l1_xyz = _fps_pallas(point_cloud, 256)
    l1_points = _sa_pallas(point_cloud, pc_t, point_cloud, l1_xyz,
                           sa1_c0_w, sa1_c0_b, sa1_c1_w, sa1_c1_b)
    l1_xyz_t = jnp.transpose(l1_xyz, (0, 2, 1))
    l1_points = _t_pallas(l1_points, l1_xyz, l1_xyz_t, p, 't1')

    l2_xyz = _fps_pallas(l1_xyz, 64)
    l2_points = _sa_pallas(l1_xyz, l1_xyz_t, l1_points, l2_xyz,
                           sa2_c0_w, sa2_c0_b, sa2_c1_w, sa2_c1_b)
    l2_xyz_t = jnp.transpose(l2_xyz, (0, 2, 1))
    l2_points = _t_pallas(l2_points, l2_xyz, l2_xyz_t, p, 't2')

    return _sa3_pallas(l2_xyz, l2_points, sa3_c0_w, sa3_c0_b, sa3_c1_w, sa3_c1_b)


# speedup vs baseline: 118.8624x; 1.0014x over previous
import functools

import jax
import jax.numpy as jnp
import numpy as np
from jax.experimental import pallas as pl
from jax.experimental.pallas import tpu as pltpu

N_KNN = 16
_BN_SCALE = np.float32(1.0 / np.sqrt(1.0 + 1e-5))
_F32 = jnp.float32
_BIG = np.float32(3.0e38)


def _fps_body(xyz_ref, out_ref, *, npoint):
    x = xyz_ref[:, 0, :]
    y = xyz_ref[:, 1, :]
    z = xyz_ref[:, 2, :]
    B, N = x.shape
    lane = jax.lax.broadcasted_iota(jnp.int32, (B, N), 1)
    lane_s = jax.lax.broadcasted_iota(jnp.int32, (B, npoint), 1)

    def body(i, carry):
        dist, far, nx, ny, nz = carry
        oh = lane == far
        cx = jnp.sum(jnp.where(oh, x, 0.0), axis=1, keepdims=True)
        cy = jnp.sum(jnp.where(oh, y, 0.0), axis=1, keepdims=True)
        cz = jnp.sum(jnp.where(oh, z, 0.0), axis=1, keepdims=True)
        sel = lane_s == i
        nx = jnp.where(sel, cx, nx)
        ny = jnp.where(sel, cy, ny)
        nz = jnp.where(sel, cz, nz)
        dx = x - cx
        dy = y - cy
        dz = z - cz
        d = (dx * dx + dy * dy) + dz * dz
        dist = jnp.minimum(dist, d)
        m = jnp.max(dist, axis=1, keepdims=True)
        far = jnp.min(jnp.where(dist == m, lane, N), axis=1, keepdims=True)
        return dist, far, nx, ny, nz

    zs = jnp.zeros((B, npoint), dtype=_F32)
    _, _, nx, ny, nz = jax.lax.fori_loop(
        0, npoint, body,
        (jnp.full((B, N), 1e10, dtype=_F32), jnp.zeros((B, 1), jnp.int32),
         zs, zs, zs))
    out_ref[:, 0, :] = nx
    out_ref[:, 1, :] = ny
    out_ref[:, 2, :] = nz


def _fps_pallas(xyz, npoint):
    B, _, N = xyz.shape
    return pl.pallas_call(
        functools.partial(_fps_body, npoint=npoint),
        out_shape=jax.ShapeDtypeStruct((B, 3, npoint), _F32),
    )(xyz)


def _sa_body(xyz_ref, xyzt_ref, pts_ref, new_ref,
             w0_ref, b0_ref, w1_ref, b1_ref, out_ref):
    xyz = xyz_ref[0]
    xyz_t = xyzt_ref[0]
    pts = pts_ref[0]
    new = new_ref[0]
    N = xyz_t.shape[0]
    S = new.shape[1]
    Cout = out_ref.shape[1]

    xt0, xt1, xt2 = xyz_t[:, 0:1], xyz_t[:, 1:2], xyz_t[:, 2:3]
    q0, q1, q2 = new[0:1, :], new[1:2, :], new[2:3, :]
    sqq = (q0 * q0 + q1 * q1) + q2 * q2
    sqx = (xt0 * xt0 + xt1 * xt1) + xt2 * xt2
    dot = jnp.dot(xyz_t, new, preferred_element_type=_F32)
    dt = (-2.0 * dot + sqq) + sqx
    sub = jax.lax.broadcasted_iota(jnp.int32, (N, S), 0)

    w0 = w0_ref[...]
    b0 = b0_ref[...]
    w1 = w1_ref[...]
    b1 = b1_ref[...]

    run = jnp.full((Cout, S), -_BIG, dtype=_F32)
    for _ in range(N_KNN):
        m = jnp.min(dt, axis=0, keepdims=True)
        idxk = jnp.min(jnp.where(dt == m, sub, N), axis=0,
                       keepdims=True)
        oh = sub == idxk
        ohf = jnp.where(oh, np.float32(1.0), np.float32(0.0))
        gx = jnp.dot(xyz, ohf, preferred_element_type=_F32)
        gp = jnp.dot(pts, ohf, preferred_element_type=_F32)
        g = jnp.concatenate([gx - new, gp], axis=0)
        h = jnp.dot(w0, g, preferred_element_type=_F32) + b0
        h = jnp.maximum(h, 0.0)
        yk = jnp.dot(w1, h, preferred_element_type=_F32) + b1
        run = jnp.maximum(run, yk)
        dt = jnp.where(oh, _BIG, dt)
    out_ref[0] = run


def _sa_pallas(xyz, xyz_t, pts, new_xyz, w0, b0, w1, b1):
    B, _, N = xyz.shape
    C = pts.shape[1]
    S = new_xyz.shape[2]
    Cout = w1.shape[0]
    Chid = w0.shape[0]
    return pl.pallas_call(
        _sa_body,
        grid=(B,),
        in_specs=[
            pl.BlockSpec((1, 3, N), lambda b: (b, 0, 0)),
            pl.BlockSpec((1, N, 3), lambda b: (b, 0, 0)),
            pl.BlockSpec((1, C, N), lambda b: (b, 0, 0)),
            pl.BlockSpec((1, 3, S), lambda b: (b, 0, 0)),
            pl.BlockSpec((Chid, C + 3), lambda b: (0, 0)),
            pl.BlockSpec((Chid, 1), lambda b: (0, 0)),
            pl.BlockSpec((Cout, Chid), lambda b: (0, 0)),
            pl.BlockSpec((Cout, 1), lambda b: (0, 0)),
        ],
        out_specs=pl.BlockSpec((1, Cout, S), lambda b: (b, 0, 0)),
        out_shape=jax.ShapeDtypeStruct((B, Cout, S), _F32),
    )(xyz, xyz_t, pts, new_xyz, w0, b0[:, None], w1, b1[:, None])


def _t_body(x_ref, pos_ref, post_ref,
            wst_ref, bst_ref, wk_ref, bk_ref, wq_ref, bq_ref, wv_ref, bv_ref,
            wp0_ref, bp0_ref, gp0_ref, hp0_ref, wp1_ref, bp1_ref,
            wa0_ref, ba0_ref, ga0_ref, ha0_ref, wa1_ref, ba1_ref,
            wend_ref, bend_ref, out_ref):
    x_in = x_ref[0]
    pos = pos_ref[0]
    pos_t = post_ref[0]
    S = pos.shape[1]
    K = N_KNN

    x = jnp.dot(wst_ref[...], x_in, preferred_element_type=_F32) + bst_ref[...]
    keya = jnp.dot(wk_ref[...], x, preferred_element_type=_F32) + bk_ref[...]
    val = jnp.dot(wv_ref[...], x, preferred_element_type=_F32) + bv_ref[...]
    qry = jnp.dot(wq_ref[...], x, preferred_element_type=_F32) + bq_ref[...]

    p0, p1, p2 = pos[0:1, :], pos[1:2, :], pos[2:3, :]
    pt0, pt1, pt2 = pos_t[:, 0:1], pos_t[:, 1:2], pos_t[:, 2:3]
    sqq = (p0 * p0 + p1 * p1) + p2 * p2
    sqx = (pt0 * pt0 + pt1 * pt1) + pt2 * pt2
    dot = jnp.dot(pos_t, pos, preferred_element_type=_F32)
    dt = (-2.0 * dot + sqq) + sqx
    sub = jax.lax.broadcasted_iota(jnp.int32, (S, S), 0)

    qks = []
    prs = []
    for _ in range(K):
        m = jnp.min(dt, axis=0, keepdims=True)
        idxk = jnp.min(jnp.where(dt == m, sub, S), axis=0, keepdims=True)
        oh = sub == idxk
        ohf = jnp.where(oh, np.float32(1.0), np.float32(0.0))
        kg = jnp.dot(keya, ohf, preferred_element_type=_F32)
        pg = jnp.dot(pos, ohf, preferred_element_type=_F32)
        qks.append(qry - kg)
        prs.append(pos - pg)
        dt = jnp.where(oh, _BIG, dt)
    qk = jnp.concatenate(qks, axis=1)
    prel = jnp.concatenate(prs, axis=1)

    pe = jnp.dot(wp0_ref[...], prel, preferred_element_type=_F32) + bp0_ref[...]
    pe = jnp.maximum(pe * gp0_ref[...] + hp0_ref[...], 0.0)
    pe = jnp.dot(wp1_ref[...], pe, preferred_element_type=_F32) + bp1_ref[...]

    a = jnp.dot(wa0_ref[...], qk + pe, preferred_element_type=_F32) + ba0_ref[...]
    a = jnp.maximum(a * ga0_ref[...] + ha0_ref[...], 0.0)
    attn = jnp.dot(wa1_ref[...], a, preferred_element_type=_F32) + ba1_ref[...]

    m0 = attn[:, 0:S]
    for k in range(1, K):
        m0 = jnp.maximum(m0, attn[:, k * S:(k + 1) * S])
    es = []
    esum = jnp.zeros_like(m0)
    for k in range(K):
        e = jnp.exp(attn[:, k * S:(k + 1) * S] - m0)
        es.append(e)
        esum = esum + e
    agg = jnp.zeros_like(m0)
    for k in range(K):
        agg = agg + (es[k] / esum) * (val + pe[:, k * S:(k + 1) * S])

    y = jnp.dot(wend_ref[...], agg, preferred_element_type=_F32) + bend_ref[...]
    out_ref[0] = y + x_in


def _t_pallas(x, pos, pos_t, p, pre):
    B, Cin, S = x.shape
    w = {n: p[pre + n] for n in (
        '_start_w', '_key_w', '_query_w', '_value_w', '_pos0_w', '_pos1_w',
        '_attn0_w', '_attn1_w', '_end_w')}
    gp0 = p[pre + '_pos_bn_g'] * _BN_SCALE
    hp0 = p[pre + '_pos_bn_b']
    ga0 = p[pre + '_attn_bn_g'] * _BN_SCALE
    ha0 = p[pre + '_attn_bn_b']

    def wspec(a):
        return pl.BlockSpec(a.shape, lambda b: tuple(0 for _ in a.shape))

    ws = [w['_start_w'], p[pre + '_start_b'][:, None],
          w['_key_w'], p[pre + '_key_b'][:, None],
          w['_query_w'], p[pre + '_query_b'][:, None],
          w['_value_w'], p[pre + '_value_b'][:, None],
          w['_pos0_w'], p[pre + '_pos0_b'][:, None],
          gp0[:, None], hp0[:, None],
          w['_pos1_w'], p[pre + '_pos1_b'][:, None],
          w['_attn0_w'], p[pre + '_attn0_b'][:, None],
          ga0[:, None], ha0[:, None],
          w['_attn1_w'], p[pre + '_attn1_b'][:, None],
          w['_end_w'], p[pre + '_end_b'][:, None]]
    return pl.pallas_call(
        _t_body,
        grid=(B,),
        in_specs=[
            pl.BlockSpec((1, Cin, S), lambda b: (b, 0, 0)),
            pl.BlockSpec((1, 3, S), lambda b: (b, 0, 0)),
            pl.BlockSpec((1, S, 3), lambda b: (b, 0, 0)),
        ] + [wspec(a) for a in ws],
        out_specs=pl.BlockSpec((1, Cin, S), lambda b: (b, 0, 0)),
        out_shape=jax.ShapeDtypeStruct((B, Cin, S), _F32),
    )(x, pos, pos_t, *ws)


def _sa3_body(xyz_ref, pts_ref, w0x_ref, w0p_ref, b0_ref, w1_ref, b1_ref,
              out_ref):
    xyz = xyz_ref[0]
    pts = pts_ref[0]
    h = (jnp.dot(w0x_ref[...], xyz, preferred_element_type=_F32)
         + jnp.dot(w0p_ref[...], pts, preferred_element_type=_F32)
         + b0_ref[...])
    h = jnp.maximum(h, 0.0)
    y = jnp.dot(w1_ref[...], h, preferred_element_type=_F32)
    y = y + b1_ref[...]
    out_ref[0] = jnp.max(y, axis=1, keepdims=True)


def _sa3_pallas(l2_xyz, l2_points, w0, b0, w1, b1):
    B = l2_xyz.shape[0]
    return pl.pallas_call(
        _sa3_body,
        grid=(B,),
        in_specs=[
            pl.BlockSpec((1, 3, 64), lambda b: (b, 0, 0)),
            pl.BlockSpec((1, 128, 64), lambda b: (b, 0, 0)),
            pl.BlockSpec((256, 3), lambda b: (0, 0)),
            pl.BlockSpec((256, 128), lambda b: (0, 0)),
            pl.BlockSpec((256, 1), lambda b: (0, 0)),
            pl.BlockSpec((512, 256), lambda b: (0, 0)),
            pl.BlockSpec((512, 1), lambda b: (0, 0)),
        ],
        out_specs=pl.BlockSpec((1, 512, 1), lambda b: (b, 0, 0)),
        out_shape=jax.ShapeDtypeStruct((B, 512, 1), _F32),
    )(l2_xyz, l2_points, w0[:, :3], w0[:, 3:], b0[:, None], w1, b1[:, None])


def kernel(point_cloud, sa1_c0_w, sa1_c0_b, sa1_c1_w, sa1_c1_b, t1_start_w, t1_start_b, t1_key_w, t1_key_b, t1_query_w, t1_query_b, t1_value_w, t1_value_b, t1_pos0_w, t1_pos0_b, t1_pos_bn_g, t1_pos_bn_b, t1_pos1_w, t1_pos1_b, t1_attn0_w, t1_attn0_b, t1_attn_bn_g, t1_attn_bn_b, t1_attn1_w, t1_attn1_b, t1_end_w, t1_end_b, sa2_c0_w, sa2_c0_b, sa2_c1_w, sa2_c1_b, t2_start_w, t2_start_b, t2_key_w, t2_key_b, t2_query_w, t2_query_b, t2_value_w, t2_value_b, t2_pos0_w, t2_pos0_b, t2_pos_bn_g, t2_pos_bn_b, t2_pos1_w, t2_pos1_b, t2_attn0_w, t2_attn0_b, t2_attn_bn_g, t2_attn_bn_b, t2_attn1_w, t2_attn1_b, t2_end_w, t2_end_b, sa3_c0_w, sa3_c0_b, sa3_c1_w, sa3_c1_b):
    p = dict(locals())
    del p['point_cloud']

    pc_t = jnp.transpose(point_cloud, (0, 2, 1))
    l1_xyz = _fps_pallas(point_cloud, 256)
    l1_points = _sa_pallas(point_cloud, pc_t, point_cloud, l1_xyz,
                           sa1_c0_w, sa1_c0_b, sa1_c1_w, sa1_c1_b)
    l1_xyz_t = jnp.transpose(l1_xyz, (0, 2, 1))
    l1_points = _t_pallas(l1_points, l1_xyz, l1_xyz_t, p, 't1')

    l2_xyz = _fps_pallas(l1_xyz, 64)
    l2_points = _sa_pallas(l1_xyz, l1_xyz_t, l1_points, l2_xyz,
                           sa2_c0_w, sa2_c0_b, sa2_c1_w, sa2_c1_b)
    l2_xyz_t = jnp.transpose(l2_xyz, (0, 2, 1))
    l2_points = _t_pallas(l2_points, l2_xyz, l2_xyz_t, p, 't2')

    return _sa3_pallas(l2_xyz, l2_points, sa3_c0_w, sa3_c0_b, sa3_c1_w, sa3_c1_b)


# speedup vs baseline: 126.1401x; 1.0612x over previous
import functools

import jax
import jax.numpy as jnp
import numpy as np
from jax import lax
from jax.experimental import pallas as pl
from jax.experimental.pallas import tpu as pltpu
from jax.experimental.pallas import tpu_sc as plsc

N_KNN = 16
_BN_SCALE = np.float32(1.0 / np.sqrt(1.0 + 1e-5))
_F32 = jnp.float32
_BIG = np.float32(3.0e38)


def _fps_body(xyz_ref, out_ref, *, npoint):
    x = xyz_ref[:, 0, :]
    y = xyz_ref[:, 1, :]
    z = xyz_ref[:, 2, :]
    B, N = x.shape
    lane = jax.lax.broadcasted_iota(jnp.int32, (B, N), 1)
    lane_s = jax.lax.broadcasted_iota(jnp.int32, (B, npoint), 1)

    def body(i, carry):
        dist, far, nx, ny, nz = carry
        oh = lane == far
        cx = jnp.sum(jnp.where(oh, x, 0.0), axis=1, keepdims=True)
        cy = jnp.sum(jnp.where(oh, y, 0.0), axis=1, keepdims=True)
        cz = jnp.sum(jnp.where(oh, z, 0.0), axis=1, keepdims=True)
        sel = lane_s == i
        nx = jnp.where(sel, cx, nx)
        ny = jnp.where(sel, cy, ny)
        nz = jnp.where(sel, cz, nz)
        dx = x - cx
        dy = y - cy
        dz = z - cz
        d = (dx * dx + dy * dy) + dz * dz
        dist = jnp.minimum(dist, d)
        m = jnp.max(dist, axis=1, keepdims=True)
        far = jnp.min(jnp.where(dist == m, lane, N), axis=1, keepdims=True)
        return dist, far, nx, ny, nz

    zs = jnp.zeros((B, npoint), dtype=_F32)
    _, _, nx, ny, nz = jax.lax.fori_loop(
        0, npoint, body,
        (jnp.full((B, N), 1e10, dtype=_F32), jnp.zeros((B, 1), jnp.int32),
         zs, zs, zs))
    out_ref[:, 0, :] = nx
    out_ref[:, 1, :] = ny
    out_ref[:, 2, :] = nz


def _fps_pallas(xyz, npoint):
    B, _, N = xyz.shape
    return pl.pallas_call(
        functools.partial(_fps_body, npoint=npoint),
        out_shape=jax.ShapeDtypeStruct((B, 3, npoint), _F32),
    )(xyz)


def _fps_sc(xyz, npoint):
    B, _, N = xyz.shape
    nchunk = N // 16
    mesh = plsc.VectorSubcoreMesh(core_axis_name="c", subcore_axis_name="s")

    @functools.partial(
        pl.kernel, mesh=mesh,
        out_type=jax.ShapeDtypeStruct((B * 8 * npoint,), _F32),
        scratch_types=[
            pltpu.VMEM((8 * N,), _F32),
            pltpu.VMEM((N,), _F32),
            pltpu.VMEM((8 * npoint,), _F32),
        ],
    )
    def k(xyz_hbm, out_hbm, xyz_v, dist_v, new_v):
        lane = lax.iota(jnp.int32, 16)
        b = lax.axis_index("s") * 2 + lax.axis_index("c")
        pltpu.sync_copy(xyz_hbm.at[pl.ds(b * (8 * N), 8 * N)], xyz_v)
        big = jnp.full((16,), 1e10, dtype=_F32)
        for j in range(nchunk):
            dist_v[pl.ds(j * 16, 16)] = big

        def body(i, far):
            fo = pl.multiple_of((far // 16) * 16, 16)
            fl = far % 16
            xs = xyz_v[pl.ds(fo, 16)]
            ys = xyz_v[pl.ds(N + fo, 16)]
            zs = xyz_v[pl.ds(2 * N + fo, 16)]
            cx_s, cy_s, cz_s = xs[0], ys[0], zs[0]
            for l in range(1, 16):
                pick = fl == l
                cx_s = jnp.where(pick, xs[l], cx_s)
                cy_s = jnp.where(pick, ys[l], cy_s)
                cz_s = jnp.where(pick, zs[l], cz_s)
            cx = jnp.full((16,), cx_s, _F32)
            cy = jnp.full((16,), cy_s, _F32)
            cz = jnp.full((16,), cz_s, _F32)
            co = pl.multiple_of((i // 16) * 16, 16)
            msk = lane == (i % 16)
            for c, cv in ((0, cx), (1, cy), (2, cz)):
                sl = pl.ds(c * npoint + co, 16)
                new_v[sl] = jnp.where(msk, cv, new_v[sl])
            rmax = jnp.full((16,), -1.0, dtype=_F32)
            ridx = jnp.zeros((16,), dtype=jnp.int32)
            for j in range(nchunk):
                o = j * 16
                dx = xyz_v[pl.ds(o, 16)] - cx
                dy = xyz_v[pl.ds(N + o, 16)] - cy
                dz = xyz_v[pl.ds(2 * N + o, 16)] - cz
                d = (dx * dx + dy * dy) + dz * dz
                nd = jnp.minimum(dist_v[pl.ds(o, 16)], d)
                dist_v[pl.ds(o, 16)] = nd
                upd = nd > rmax
                rmax = jnp.where(upd, nd, rmax)
                ridx = jnp.where(upd, lane + o, ridx)
            best_v = rmax[0]
            best_i = ridx[0]
            for l in range(1, 16):
                v = rmax[l]
                ix = ridx[l]
                better = jnp.logical_or(
                    v > best_v, jnp.logical_and(v == best_v, ix < best_i))
                best_v = jnp.where(better, v, best_v)
                best_i = jnp.where(better, ix, best_i)
            return best_i

        lax.fori_loop(0, npoint, body, jnp.int32(0))
        pltpu.sync_copy(new_v, out_hbm.at[pl.ds(b * (8 * npoint), 8 * npoint)])

    xyz8 = jnp.pad(xyz, ((0, 0), (0, 5), (0, 0))).reshape(B * 8 * N)
    return k(xyz8).reshape(B, 8, npoint)[:, :3]


def _sa_body(xyz_ref, xyzt_ref, pts_ref, new_ref,
             w0_ref, b0_ref, w1_ref, b1_ref, out_ref):
    xyz = xyz_ref[0]
    xyz_t = xyzt_ref[0]
    pts = pts_ref[0]
    new = new_ref[0]
    N = xyz_t.shape[0]
    S = new.shape[1]
    Cout = out_ref.shape[1]

    xt0, xt1, xt2 = xyz_t[:, 0:1], xyz_t[:, 1:2], xyz_t[:, 2:3]
    q0, q1, q2 = new[0:1, :], new[1:2, :], new[2:3, :]
    sqq = (q0 * q0 + q1 * q1) + q2 * q2
    sqx = (xt0 * xt0 + xt1 * xt1) + xt2 * xt2
    dot = jnp.dot(xyz_t, new, preferred_element_type=_F32)
    dt = (-2.0 * dot + sqq) + sqx
    sub = jax.lax.broadcasted_iota(jnp.int32, (N, S), 0)

    w0 = w0_ref[...]
    b0 = b0_ref[...]
    w1 = w1_ref[...]
    b1 = b1_ref[...]

    run = jnp.full((Cout, S), -_BIG, dtype=_F32)
    for _ in range(N_KNN):
        m = jnp.min(dt, axis=0, keepdims=True)
        idxk = jnp.min(jnp.where(dt == m, sub, N), axis=0,
                       keepdims=True)
        oh = sub == idxk
        ohf = jnp.where(oh, np.float32(1.0), np.float32(0.0))
        gx = jnp.dot(xyz, ohf, preferred_element_type=_F32)
        gp = jnp.dot(pts, ohf, preferred_element_type=_F32)
        g = jnp.concatenate([gx - new, gp], axis=0)
        h = jnp.dot(w0, g, preferred_element_type=_F32) + b0
        h = jnp.maximum(h, 0.0)
        yk = jnp.dot(w1, h, preferred_element_type=_F32) + b1
        run = jnp.maximum(run, yk)
        dt = jnp.where(oh, _BIG, dt)
    out_ref[0] = run


def _sa_pallas(xyz, xyz_t, pts, new_xyz, w0, b0, w1, b1):
    B, _, N = xyz.shape
    C = pts.shape[1]
    S = new_xyz.shape[2]
    Cout = w1.shape[0]
    Chid = w0.shape[0]
    return pl.pallas_call(
        _sa_body,
        grid=(B,),
        in_specs=[
            pl.BlockSpec((1, 3, N), lambda b: (b, 0, 0)),
            pl.BlockSpec((1, N, 3), lambda b: (b, 0, 0)),
            pl.BlockSpec((1, C, N), lambda b: (b, 0, 0)),
            pl.BlockSpec((1, 3, S), lambda b: (b, 0, 0)),
            pl.BlockSpec((Chid, C + 3), lambda b: (0, 0)),
            pl.BlockSpec((Chid, 1), lambda b: (0, 0)),
            pl.BlockSpec((Cout, Chid), lambda b: (0, 0)),
            pl.BlockSpec((Cout, 1), lambda b: (0, 0)),
        ],
        out_specs=pl.BlockSpec((1, Cout, S), lambda b: (b, 0, 0)),
        out_shape=jax.ShapeDtypeStruct((B, Cout, S), _F32),
    )(xyz, xyz_t, pts, new_xyz, w0, b0[:, None], w1, b1[:, None])


def _t_body(x_ref, pos_ref, post_ref,
            wst_ref, bst_ref, wk_ref, bk_ref, wq_ref, bq_ref, wv_ref, bv_ref,
            wp0_ref, bp0_ref, gp0_ref, hp0_ref, wp1_ref, bp1_ref,
            wa0_ref, ba0_ref, ga0_ref, ha0_ref, wa1_ref, ba1_ref,
            wend_ref, bend_ref, out_ref):
    x_in = x_ref[0]
    pos = pos_ref[0]
    pos_t = post_ref[0]
    S = pos.shape[1]
    K = N_KNN

    x = jnp.dot(wst_ref[...], x_in, preferred_element_type=_F32) + bst_ref[...]
    keya = jnp.dot(wk_ref[...], x, preferred_element_type=_F32) + bk_ref[...]
    val = jnp.dot(wv_ref[...], x, preferred_element_type=_F32) + bv_ref[...]
    qry = jnp.dot(wq_ref[...], x, preferred_element_type=_F32) + bq_ref[...]

    p0, p1, p2 = pos[0:1, :], pos[1:2, :], pos[2:3, :]
    pt0, pt1, pt2 = pos_t[:, 0:1], pos_t[:, 1:2], pos_t[:, 2:3]
    sqq = (p0 * p0 + p1 * p1) + p2 * p2
    sqx = (pt0 * pt0 + pt1 * pt1) + pt2 * pt2
    dot = jnp.dot(pos_t, pos, preferred_element_type=_F32)
    dt = (-2.0 * dot + sqq) + sqx
    sub = jax.lax.broadcasted_iota(jnp.int32, (S, S), 0)

    qks = []
    prs = []
    for _ in range(K):
        m = jnp.min(dt, axis=0, keepdims=True)
        idxk = jnp.min(jnp.where(dt == m, sub, S), axis=0, keepdims=True)
        oh = sub == idxk
        ohf = jnp.where(oh, np.float32(1.0), np.float32(0.0))
        kg = jnp.dot(keya, ohf, preferred_element_type=_F32)
        pg = jnp.dot(pos, ohf, preferred_element_type=_F32)
        qks.append(qry - kg)
        prs.append(pos - pg)
        dt = jnp.where(oh, _BIG, dt)
    qk = jnp.concatenate(qks, axis=1)
    prel = jnp.concatenate(prs, axis=1)

    pe = jnp.dot(wp0_ref[...], prel, preferred_element_type=_F32) + bp0_ref[...]
    pe = jnp.maximum(pe * gp0_ref[...] + hp0_ref[...], 0.0)
    pe = jnp.dot(wp1_ref[...], pe, preferred_element_type=_F32) + bp1_ref[...]

    a = jnp.dot(wa0_ref[...], qk + pe, preferred_element_type=_F32) + ba0_ref[...]
    a = jnp.maximum(a * ga0_ref[...] + ha0_ref[...], 0.0)
    attn = jnp.dot(wa1_ref[...], a, preferred_element_type=_F32) + ba1_ref[...]

    m0 = attn[:, 0:S]
    for k in range(1, K):
        m0 = jnp.maximum(m0, attn[:, k * S:(k + 1) * S])
    es = []
    esum = jnp.zeros_like(m0)
    for k in range(K):
        e = jnp.exp(attn[:, k * S:(k + 1) * S] - m0)
        es.append(e)
        esum = esum + e
    agg = jnp.zeros_like(m0)
    for k in range(K):
        agg = agg + (es[k] / esum) * (val + pe[:, k * S:(k + 1) * S])

    y = jnp.dot(wend_ref[...], agg, preferred_element_type=_F32) + bend_ref[...]
    out_ref[0] = y + x_in


def _t_pallas(x, pos, pos_t, p, pre):
    B, Cin, S = x.shape
    w = {n: p[pre + n] for n in (
        '_start_w', '_key_w', '_query_w', '_value_w', '_pos0_w', '_pos1_w',
        '_attn0_w', '_attn1_w', '_end_w')}
    gp0 = p[pre + '_pos_bn_g'] * _BN_SCALE
    hp0 = p[pre + '_pos_bn_b']
    ga0 = p[pre + '_attn_bn_g'] * _BN_SCALE
    ha0 = p[pre + '_attn_bn_b']

    def wspec(a):
        return pl.BlockSpec(a.shape, lambda b: tuple(0 for _ in a.shape))

    ws = [w['_start_w'], p[pre + '_start_b'][:, None],
          w['_key_w'], p[pre + '_key_b'][:, None],
          w['_query_w'], p[pre + '_query_b'][:, None],
          w['_value_w'], p[pre + '_value_b'][:, None],
          w['_pos0_w'], p[pre + '_pos0_b'][:, None],
          gp0[:, None], hp0[:, None],
          w['_pos1_w'], p[pre + '_pos1_b'][:, None],
          w['_attn0_w'], p[pre + '_attn0_b'][:, None],
          ga0[:, None], ha0[:, None],
          w['_attn1_w'], p[pre + '_attn1_b'][:, None],
          w['_end_w'], p[pre + '_end_b'][:, None]]
    return pl.pallas_call(
        _t_body,
        grid=(B,),
        in_specs=[
            pl.BlockSpec((1, Cin, S), lambda b: (b, 0, 0)),
            pl.BlockSpec((1, 3, S), lambda b: (b, 0, 0)),
            pl.BlockSpec((1, S, 3), lambda b: (b, 0, 0)),
        ] + [wspec(a) for a in ws],
        out_specs=pl.BlockSpec((1, Cin, S), lambda b: (b, 0, 0)),
        out_shape=jax.ShapeDtypeStruct((B, Cin, S), _F32),
    )(x, pos, pos_t, *ws)


def _sa3_body(xyz_ref, pts_ref, w0x_ref, w0p_ref, b0_ref, w1_ref, b1_ref,
              out_ref):
    xyz = xyz_ref[0]
    pts = pts_ref[0]
    h = (jnp.dot(w0x_ref[...], xyz, preferred_element_type=_F32)
         + jnp.dot(w0p_ref[...], pts, preferred_element_type=_F32)
         + b0_ref[...])
    h = jnp.maximum(h, 0.0)
    y = jnp.dot(w1_ref[...], h, preferred_element_type=_F32)
    y = y + b1_ref[...]
    out_ref[0] = jnp.max(y, axis=1, keepdims=True)


def _sa3_pallas(l2_xyz, l2_points, w0, b0, w1, b1):
    B = l2_xyz.shape[0]
    return pl.pallas_call(
        _sa3_body,
        grid=(B,),
        in_specs=[
            pl.BlockSpec((1, 3, 64), lambda b: (b, 0, 0)),
            pl.BlockSpec((1, 128, 64), lambda b: (b, 0, 0)),
            pl.BlockSpec((256, 3), lambda b: (0, 0)),
            pl.BlockSpec((256, 128), lambda b: (0, 0)),
            pl.BlockSpec((256, 1), lambda b: (0, 0)),
            pl.BlockSpec((512, 256), lambda b: (0, 0)),
            pl.BlockSpec((512, 1), lambda b: (0, 0)),
        ],
        out_specs=pl.BlockSpec((1, 512, 1), lambda b: (b, 0, 0)),
        out_shape=jax.ShapeDtypeStruct((B, 512, 1), _F32),
    )(l2_xyz, l2_points, w0[:, :3], w0[:, 3:], b0[:, None], w1, b1[:, None])


def kernel(point_cloud, sa1_c0_w, sa1_c0_b, sa1_c1_w, sa1_c1_b, t1_start_w, t1_start_b, t1_key_w, t1_key_b, t1_query_w, t1_query_b, t1_value_w, t1_value_b, t1_pos0_w, t1_pos0_b, t1_pos_bn_g, t1_pos_bn_b, t1_pos1_w, t1_pos1_b, t1_attn0_w, t1_attn0_b, t1_attn_bn_g, t1_attn_bn_b, t1_attn1_w, t1_attn1_b, t1_end_w, t1_end_b, sa2_c0_w, sa2_c0_b, sa2_c1_w, sa2_c1_b, t2_start_w, t2_start_b, t2_key_w, t2_key_b, t2_query_w, t2_query_b, t2_value_w, t2_value_b, t2_pos0_w, t2_pos0_b, t2_pos_bn_g, t2_pos_bn_b, t2_pos1_w, t2_pos1_b, t2_attn0_w, t2_attn0_b, t2_attn_bn_g, t2_attn_bn_b, t2_attn1_w, t2_attn1_b, t2_end_w, t2_end_b, sa3_c0_w, sa3_c0_b, sa3_c1_w, sa3_c1_b):
    p = dict(locals())
    del p['point_cloud']

    pc_t = jnp.transpose(point_cloud, (0, 2, 1))
    l1_xyz = _fps_sc(point_cloud, 256)
    l1_points = _sa_pallas(point_cloud, pc_t, point_cloud, l1_xyz,
                           sa1_c0_w, sa1_c0_b, sa1_c1_w, sa1_c1_b)
    l1_xyz_t = jnp.transpose(l1_xyz, (0, 2, 1))
    l1_points = _t_pallas(l1_points, l1_xyz, l1_xyz_t, p, 't1')

    l2_xyz = _fps_sc(l1_xyz, 64)
    l2_points = _sa_pallas(l1_xyz, l1_xyz_t, l1_points, l2_xyz,
                           sa2_c0_w, sa2_c0_b, sa2_c1_w, sa2_c1_b)
    l2_xyz_t = jnp.transpose(l2_xyz, (0, 2, 1))
    l2_points = _t_pallas(l2_points, l2_xyz, l2_xyz_t, p, 't2')

    return _sa3_pallas(l2_xyz, l2_points, sa3_c0_w, sa3_c0_b, sa3_c1_w, sa3_c1_b)
